# Initial kernel scaffold; baseline (speedup 1.0000x reference)
import jax
import jax.numpy as jnp
from jax.experimental import pallas as pl


def kernel(xyz1, xyz2, features1, features2, W1, g1, b1, W2, g2, b2):
    raise NotImplementedError("write your pallas kernel here")


# speedup vs baseline: 9.6114x; 9.6114x over previous
import functools

import jax
import jax.numpy as jnp
from jax.experimental import pallas as pl


def _stage_a(g2_ref, g1_ref, x1sq_ref, x2sq_ref, f1_ref, f2_ref,
             w1a_ref, w1b_ref, h1_ref, s_ref, q_ref, *, n2):
    g2 = g2_ref[0]
    g1 = g1_ref[0]
    cross = jax.lax.dot_general(g2, g1, (((0,), (0,)), ((), ())),
                                preferred_element_type=jnp.float32)
    x1sq = x1sq_ref[0, 0, :]
    d2 = (x2sq_ref[0] + x1sq[None, :]) - 2.0 * cross
    dist = jnp.sqrt(jnp.maximum(d2, 0.0))

    iota = jax.lax.broadcasted_iota(jnp.int32, dist.shape, 0)
    cur = dist
    mins, amins = [], []
    for _ in range(3):
        m = jnp.min(cur, axis=0)
        am = jnp.min(jnp.where(cur == m[None, :], iota, n2), axis=0)
        mins.append(m)
        amins.append(am)
        cur = jnp.where(iota == am[None, :], jnp.float32(3e38), cur)

    ws = [1.0 / (m + 1e-8) for m in mins]
    norm = ws[0] + ws[1] + ws[2]
    ws = [w / norm for w in ws]

    st = jnp.where(iota == amins[0][None, :], ws[0][None, :], 0.0)
    st = st + jnp.where(iota == amins[1][None, :], ws[1][None, :], 0.0)
    st = st + jnp.where(iota == amins[2][None, :], ws[2][None, :], 0.0)

    interp = jnp.dot(f2_ref[0], st, preferred_element_type=jnp.float32,
                    precision=jax.lax.Precision.HIGHEST)
    h1 = (jnp.dot(w1a_ref[...], f1_ref[0], preferred_element_type=jnp.float32,
                    precision=jax.lax.Precision.HIGHEST)
          + jnp.dot(w1b_ref[...], interp, preferred_element_type=jnp.float32,
                      precision=jax.lax.Precision.HIGHEST))
    h1_ref[0] = h1

    psum = jnp.sum(h1, axis=1, keepdims=True)
    psq = jnp.sum(h1 * h1, axis=1, keepdims=True)
    first = (pl.program_id(0) == 0) & (pl.program_id(1) == 0)

    @pl.when(first)
    def _():
        s_ref[...] = psum
        q_ref[...] = psq

    @pl.when(jnp.logical_not(first))
    def _():
        s_ref[...] += psum
        q_ref[...] += psq


def _stage_b(h1_ref, s_ref, q_ref, gam_ref, bet_ref, w2_ref,
             h2_ref, s2_ref, q2_ref, *, count):
    mean = s_ref[...] / count
    var = q_ref[...] / count - mean * mean
    scale = gam_ref[...] * jax.lax.rsqrt(var + 1e-5)
    shift = bet_ref[...] - mean * scale
    act = jnp.maximum(scale * h1_ref[0] + shift, 0.0)
    h2 = jnp.dot(w2_ref[...], act, preferred_element_type=jnp.float32,
                 precision=jax.lax.Precision.HIGHEST)
    h2_ref[0] = h2

    psum = jnp.sum(h2, axis=1, keepdims=True)
    psq = jnp.sum(h2 * h2, axis=1, keepdims=True)
    first = (pl.program_id(0) == 0) & (pl.program_id(1) == 0)

    @pl.when(first)
    def _():
        s2_ref[...] = psum
        q2_ref[...] = psq

    @pl.when(jnp.logical_not(first))
    def _():
        s2_ref[...] += psum
        q2_ref[...] += psq


def _stage_c(h2_ref, s_ref, q_ref, gam_ref, bet_ref, out_ref, *, count):
    mean = s_ref[...] / count
    var = q_ref[...] / count - mean * mean
    scale = gam_ref[...] * jax.lax.rsqrt(var + 1e-5)
    shift = bet_ref[...] - mean * scale
    out_ref[0] = jnp.maximum(scale * h2_ref[0] + shift, 0.0)


def kernel(xyz1, xyz2, features1, features2, W1, g1, b1, W2, g2, b2):
    B, N1, _ = xyz1.shape
    N2 = xyz2.shape[1]
    C1 = features1.shape[1]
    C2 = features2.shape[1]
    CO1 = W1.shape[0]
    CO2 = W2.shape[0]
    TN = 256
    NT = N1 // TN
    f32 = jnp.float32

    bf16 = jnp.bfloat16
    x1t = jnp.transpose(xyz1, (0, 2, 1)).astype(bf16)
    x2t = jnp.transpose(xyz2, (0, 2, 1)).astype(bf16)
    x1sq = jnp.sum(xyz1 ** 2, axis=-1)
    x2sq = jnp.sum(xyz2 ** 2, axis=-1)
    zer1 = jnp.zeros((B, 13, N1), bf16)
    zer2 = jnp.zeros((B, 13, N2), bf16)
    G1 = jnp.concatenate([x1t, zer1], axis=1)
    G2 = jnp.concatenate([x2t, zer2], axis=1)
    X1SQ = x1sq[:, None, :]
    X2SQ = x2sq[:, :, None]

    W1a = W1[:, :C1]
    W1b = W1[:, C1:]

    grid = (B, NT)
    h1, s1, q1 = pl.pallas_call(
        functools.partial(_stage_a, n2=N2),
        grid=grid,
        in_specs=[
            pl.BlockSpec((1, 16, N2), lambda b, t: (b, 0, 0)),
            pl.BlockSpec((1, 16, TN), lambda b, t: (b, 0, t)),
            pl.BlockSpec((1, 1, TN), lambda b, t: (b, 0, t)),
            pl.BlockSpec((1, N2, 1), lambda b, t: (b, 0, 0)),
            pl.BlockSpec((1, C1, TN), lambda b, t: (b, 0, t)),
            pl.BlockSpec((1, C2, N2), lambda b, t: (b, 0, 0)),
            pl.BlockSpec((CO1, C1), lambda b, t: (0, 0)),
            pl.BlockSpec((CO1, C2), lambda b, t: (0, 0)),
        ],
        out_specs=[
            pl.BlockSpec((1, CO1, TN), lambda b, t: (b, 0, t)),
            pl.BlockSpec((CO1, 1), lambda b, t: (0, 0)),
            pl.BlockSpec((CO1, 1), lambda b, t: (0, 0)),
        ],
        out_shape=[
            jax.ShapeDtypeStruct((B, CO1, N1), f32),
            jax.ShapeDtypeStruct((CO1, 1), f32),
            jax.ShapeDtypeStruct((CO1, 1), f32),
        ],
    )(G2, G1, X1SQ, X2SQ, features1, features2, W1a, W1b)

    count = float(B * N1)
    h2, s2, q2 = pl.pallas_call(
        functools.partial(_stage_b, count=count),
        grid=grid,
        in_specs=[
            pl.BlockSpec((1, CO1, TN), lambda b, t: (b, 0, t)),
            pl.BlockSpec((CO1, 1), lambda b, t: (0, 0)),
            pl.BlockSpec((CO1, 1), lambda b, t: (0, 0)),
            pl.BlockSpec((CO1, 1), lambda b, t: (0, 0)),
            pl.BlockSpec((CO1, 1), lambda b, t: (0, 0)),
            pl.BlockSpec((CO2, CO1), lambda b, t: (0, 0)),
        ],
        out_specs=[
            pl.BlockSpec((1, CO2, TN), lambda b, t: (b, 0, t)),
            pl.BlockSpec((CO2, 1), lambda b, t: (0, 0)),
            pl.BlockSpec((CO2, 1), lambda b, t: (0, 0)),
        ],
        out_shape=[
            jax.ShapeDtypeStruct((B, CO2, N1), f32),
            jax.ShapeDtypeStruct((CO2, 1), f32),
            jax.ShapeDtypeStruct((CO2, 1), f32),
        ],
    )(h1, s1, q1, g1.reshape(CO1, 1), b1.reshape(CO1, 1), W2)

    out = pl.pallas_call(
        functools.partial(_stage_c, count=count),
        grid=grid,
        in_specs=[
            pl.BlockSpec((1, CO2, TN), lambda b, t: (b, 0, t)),
            pl.BlockSpec((CO2, 1), lambda b, t: (0, 0)),
            pl.BlockSpec((CO2, 1), lambda b, t: (0, 0)),
            pl.BlockSpec((CO2, 1), lambda b, t: (0, 0)),
            pl.BlockSpec((CO2, 1), lambda b, t: (0, 0)),
        ],
        out_specs=pl.BlockSpec((1, CO2, TN), lambda b, t: (b, 0, t)),
        out_shape=jax.ShapeDtypeStruct((B, CO2, N1), f32),
    )(h2, s2, q2, g2.reshape(CO2, 1), b2.reshape(CO2, 1))

    return out


# speedup vs baseline: 14.3984x; 1.4980x over previous
import functools

import jax
import jax.numpy as jnp
from jax.experimental import pallas as pl


def _stage_a(g2_ref, g1_ref, x1sq_ref, x2sq_ref, f1_ref, f2_ref,
             w1a_ref, w1b_ref, h1_ref, s_ref, q_ref, *, n2):
    g2 = g2_ref[0]
    g1 = g1_ref[0]
    cross = jax.lax.dot_general(g2, g1, (((0,), (0,)), ((), ())),
                                preferred_element_type=jnp.float32)
    x1sq = x1sq_ref[0, 0, :]
    d2 = (x2sq_ref[0] + x1sq[None, :]) - 2.0 * cross
    dist = jnp.sqrt(jnp.maximum(d2, 0.0))

    iota = jax.lax.broadcasted_iota(jnp.int32, dist.shape, 0)
    cur = dist
    mins, amins = [], []
    for _ in range(3):
        m = jnp.min(cur, axis=0)
        am = jnp.min(jnp.where(cur == m[None, :], iota, n2), axis=0)
        mins.append(m)
        amins.append(am)
        cur = jnp.where(iota == am[None, :], jnp.float32(3e38), cur)

    ws = [1.0 / (m + 1e-8) for m in mins]
    norm = ws[0] + ws[1] + ws[2]
    ws = [w / norm for w in ws]

    st = jnp.where(iota == amins[0][None, :], ws[0][None, :], 0.0)
    st = st + jnp.where(iota == amins[1][None, :], ws[1][None, :], 0.0)
    st = st + jnp.where(iota == amins[2][None, :], ws[2][None, :], 0.0)

    interp = jnp.dot(f2_ref[0], st, preferred_element_type=jnp.float32)
    h1 = (jnp.dot(w1a_ref[...], f1_ref[0], preferred_element_type=jnp.float32)
          + jnp.dot(w1b_ref[...], interp, preferred_element_type=jnp.float32))
    h1_ref[0] = h1

    psum = jnp.sum(h1, axis=1, keepdims=True)
    psq = jnp.sum(h1 * h1, axis=1, keepdims=True)
    first = (pl.program_id(0) == 0) & (pl.program_id(1) == 0)

    @pl.when(first)
    def _():
        s_ref[...] = psum
        q_ref[...] = psq

    @pl.when(jnp.logical_not(first))
    def _():
        s_ref[...] += psum
        q_ref[...] += psq


def _stage_b(h1_ref, s_ref, q_ref, gam_ref, bet_ref, w2_ref,
             h2_ref, s2_ref, q2_ref, *, count):
    mean = s_ref[...] / count
    var = q_ref[...] / count - mean * mean
    scale = gam_ref[...] * jax.lax.rsqrt(var + 1e-5)
    shift = bet_ref[...] - mean * scale
    act = jnp.maximum(scale * h1_ref[0] + shift, 0.0)
    h2 = jnp.dot(w2_ref[...], act, preferred_element_type=jnp.float32)
    h2_ref[0] = h2

    psum = jnp.sum(h2, axis=1, keepdims=True)
    psq = jnp.sum(h2 * h2, axis=1, keepdims=True)
    first = (pl.program_id(0) == 0) & (pl.program_id(1) == 0)

    @pl.when(first)
    def _():
        s2_ref[...] = psum
        q2_ref[...] = psq

    @pl.when(jnp.logical_not(first))
    def _():
        s2_ref[...] += psum
        q2_ref[...] += psq


def _stage_c(h2_ref, s_ref, q_ref, gam_ref, bet_ref, out_ref, *, count):
    mean = s_ref[...] / count
    var = q_ref[...] / count - mean * mean
    scale = gam_ref[...] * jax.lax.rsqrt(var + 1e-5)
    shift = bet_ref[...] - mean * scale
    out_ref[0] = jnp.maximum(scale * h2_ref[0] + shift, 0.0)


def kernel(xyz1, xyz2, features1, features2, W1, g1, b1, W2, g2, b2):
    B, N1, _ = xyz1.shape
    N2 = xyz2.shape[1]
    C1 = features1.shape[1]
    C2 = features2.shape[1]
    CO1 = W1.shape[0]
    CO2 = W2.shape[0]
    TN = 256
    NT = N1 // TN
    f32 = jnp.float32

    bf16 = jnp.bfloat16
    x1t = jnp.transpose(xyz1, (0, 2, 1)).astype(bf16)
    x2t = jnp.transpose(xyz2, (0, 2, 1)).astype(bf16)
    x1sq = jnp.sum(xyz1 ** 2, axis=-1)
    x2sq = jnp.sum(xyz2 ** 2, axis=-1)
    zer1 = jnp.zeros((B, 13, N1), bf16)
    zer2 = jnp.zeros((B, 13, N2), bf16)
    G1 = jnp.concatenate([x1t, zer1], axis=1)
    G2 = jnp.concatenate([x2t, zer2], axis=1)
    X1SQ = x1sq[:, None, :]
    X2SQ = x2sq[:, :, None]

    W1a = W1[:, :C1]
    W1b = W1[:, C1:]

    grid = (B, NT)
    h1, s1, q1 = pl.pallas_call(
        functools.partial(_stage_a, n2=N2),
        grid=grid,
        in_specs=[
            pl.BlockSpec((1, 16, N2), lambda b, t: (b, 0, 0)),
            pl.BlockSpec((1, 16, TN), lambda b, t: (b, 0, t)),
            pl.BlockSpec((1, 1, TN), lambda b, t: (b, 0, t)),
            pl.BlockSpec((1, N2, 1), lambda b, t: (b, 0, 0)),
            pl.BlockSpec((1, C1, TN), lambda b, t: (b, 0, t)),
            pl.BlockSpec((1, C2, N2), lambda b, t: (b, 0, 0)),
            pl.BlockSpec((CO1, C1), lambda b, t: (0, 0)),
            pl.BlockSpec((CO1, C2), lambda b, t: (0, 0)),
        ],
        out_specs=[
            pl.BlockSpec((1, CO1, TN), lambda b, t: (b, 0, t)),
            pl.BlockSpec((CO1, 1), lambda b, t: (0, 0)),
            pl.BlockSpec((CO1, 1), lambda b, t: (0, 0)),
        ],
        out_shape=[
            jax.ShapeDtypeStruct((B, CO1, N1), f32),
            jax.ShapeDtypeStruct((CO1, 1), f32),
            jax.ShapeDtypeStruct((CO1, 1), f32),
        ],
    )(G2, G1, X1SQ, X2SQ, features1, features2, W1a, W1b)

    count = float(B * N1)
    h2, s2, q2 = pl.pallas_call(
        functools.partial(_stage_b, count=count),
        grid=grid,
        in_specs=[
            pl.BlockSpec((1, CO1, TN), lambda b, t: (b, 0, t)),
            pl.BlockSpec((CO1, 1), lambda b, t: (0, 0)),
            pl.BlockSpec((CO1, 1), lambda b, t: (0, 0)),
            pl.BlockSpec((CO1, 1), lambda b, t: (0, 0)),
            pl.BlockSpec((CO1, 1), lambda b, t: (0, 0)),
            pl.BlockSpec((CO2, CO1), lambda b, t: (0, 0)),
        ],
        out_specs=[
            pl.BlockSpec((1, CO2, TN), lambda b, t: (b, 0, t)),
            pl.BlockSpec((CO2, 1), lambda b, t: (0, 0)),
            pl.BlockSpec((CO2, 1), lambda b, t: (0, 0)),
        ],
        out_shape=[
            jax.ShapeDtypeStruct((B, CO2, N1), f32),
            jax.ShapeDtypeStruct((CO2, 1), f32),
            jax.ShapeDtypeStruct((CO2, 1), f32),
        ],
    )(h1, s1, q1, g1.reshape(CO1, 1), b1.reshape(CO1, 1), W2)

    out = pl.pallas_call(
        functools.partial(_stage_c, count=count),
        grid=grid,
        in_specs=[
            pl.BlockSpec((1, CO2, TN), lambda b, t: (b, 0, t)),
            pl.BlockSpec((CO2, 1), lambda b, t: (0, 0)),
            pl.BlockSpec((CO2, 1), lambda b, t: (0, 0)),
            pl.BlockSpec((CO2, 1), lambda b, t: (0, 0)),
            pl.BlockSpec((CO2, 1), lambda b, t: (0, 0)),
        ],
        out_specs=pl.BlockSpec((1, CO2, TN), lambda b, t: (b, 0, t)),
        out_shape=jax.ShapeDtypeStruct((B, CO2, N1), f32),
    )(h2, s2, q2, g2.reshape(CO2, 1), b2.reshape(CO2, 1))

    return out


# speedup vs baseline: 14.6939x; 1.0205x over previous
import functools

import jax
import jax.numpy as jnp
from jax.experimental import pallas as pl
from jax.experimental.pallas import tpu as pltpu


def _stage_a(g2_ref, g1_ref, x1sq_ref, x2sq_ref, f1_ref, f2_ref,
             w1a_ref, w1b_ref, h1_ref, s_ref, q_ref, *, n2):
    g2 = g2_ref[0]
    g1 = g1_ref[0]
    cross = jax.lax.dot_general(g2, g1, (((0,), (0,)), ((), ())),
                                preferred_element_type=jnp.float32)
    x1sq = x1sq_ref[0, 0, :]
    d2 = (x2sq_ref[0] + x1sq[None, :]) - 2.0 * cross

    iota = jax.lax.broadcasted_iota(jnp.int32, d2.shape, 0)
    cur = d2
    mins, amins = [], []
    for _ in range(3):
        m = jnp.min(cur, axis=0)
        am = jnp.min(jnp.where(cur == m[None, :], iota, n2), axis=0)
        mins.append(m)
        amins.append(am)
        cur = jnp.where(iota == am[None, :], jnp.float32(3e38), cur)

    ws = [1.0 / (jnp.sqrt(jnp.maximum(m, 0.0)) + 1e-8) for m in mins]
    norm = ws[0] + ws[1] + ws[2]
    ws = [w / norm for w in ws]

    st = jnp.where(iota == amins[0][None, :], ws[0][None, :], 0.0)
    st = st + jnp.where(iota == amins[1][None, :], ws[1][None, :], 0.0)
    st = st + jnp.where(iota == amins[2][None, :], ws[2][None, :], 0.0)

    interp = jnp.dot(f2_ref[0], st, preferred_element_type=jnp.float32)
    h1 = (jnp.dot(w1a_ref[...], f1_ref[0], preferred_element_type=jnp.float32)
          + jnp.dot(w1b_ref[...], interp, preferred_element_type=jnp.float32))
    h1_ref[0] = h1

    psum = jnp.sum(h1, axis=1, keepdims=True)
    psq = jnp.sum(h1 * h1, axis=1, keepdims=True)
    first = pl.program_id(1) == 0

    @pl.when(first)
    def _():
        s_ref[0] = psum
        q_ref[0] = psq

    @pl.when(jnp.logical_not(first))
    def _():
        s_ref[0] += psum
        q_ref[0] += psq


def _stage_b(h1_ref, s_ref, q_ref, gam_ref, bet_ref, w2_ref,
             h2_ref, s2_ref, q2_ref, *, count):
    mean = jnp.sum(s_ref[...], axis=0) / count
    var = jnp.sum(q_ref[...], axis=0) / count - mean * mean
    scale = gam_ref[...] * jax.lax.rsqrt(var + 1e-5)
    shift = bet_ref[...] - mean * scale
    act = jnp.maximum(scale * h1_ref[0] + shift, 0.0)
    h2 = jnp.dot(w2_ref[...], act, preferred_element_type=jnp.float32)
    h2_ref[0] = h2

    psum = jnp.sum(h2, axis=1, keepdims=True)
    psq = jnp.sum(h2 * h2, axis=1, keepdims=True)
    first = pl.program_id(1) == 0

    @pl.when(first)
    def _():
        s2_ref[0] = psum
        q2_ref[0] = psq

    @pl.when(jnp.logical_not(first))
    def _():
        s2_ref[0] += psum
        q2_ref[0] += psq


def _stage_c(h2_ref, s_ref, q_ref, gam_ref, bet_ref, out_ref, *, count):
    mean = jnp.sum(s_ref[...], axis=0) / count
    var = jnp.sum(q_ref[...], axis=0) / count - mean * mean
    scale = gam_ref[...] * jax.lax.rsqrt(var + 1e-5)
    shift = bet_ref[...] - mean * scale
    out_ref[0] = jnp.maximum(scale * h2_ref[0] + shift, 0.0)


def kernel(xyz1, xyz2, features1, features2, W1, g1, b1, W2, g2, b2):
    B, N1, _ = xyz1.shape
    N2 = xyz2.shape[1]
    C1 = features1.shape[1]
    C2 = features2.shape[1]
    CO1 = W1.shape[0]
    CO2 = W2.shape[0]
    TN = 256
    NT = N1 // TN
    f32 = jnp.float32

    bf16 = jnp.bfloat16
    x1t = jnp.transpose(xyz1, (0, 2, 1)).astype(bf16)
    x2t = jnp.transpose(xyz2, (0, 2, 1)).astype(bf16)
    x1sq = jnp.sum(xyz1 ** 2, axis=-1)
    x2sq = jnp.sum(xyz2 ** 2, axis=-1)
    zer1 = jnp.zeros((B, 13, N1), bf16)
    zer2 = jnp.zeros((B, 13, N2), bf16)
    G1 = jnp.concatenate([x1t, zer1], axis=1)
    G2 = jnp.concatenate([x2t, zer2], axis=1)
    X1SQ = x1sq[:, None, :]
    X2SQ = x2sq[:, :, None]

    W1a = W1[:, :C1]
    W1b = W1[:, C1:]

    grid = (B, NT)
    params = pltpu.CompilerParams(
        dimension_semantics=("parallel", "arbitrary"))
    h1, s1, q1 = pl.pallas_call(
        functools.partial(_stage_a, n2=N2),
        grid=grid,
        in_specs=[
            pl.BlockSpec((1, 16, N2), lambda b, t: (b, 0, 0)),
            pl.BlockSpec((1, 16, TN), lambda b, t: (b, 0, t)),
            pl.BlockSpec((1, 1, TN), lambda b, t: (b, 0, t)),
            pl.BlockSpec((1, N2, 1), lambda b, t: (b, 0, 0)),
            pl.BlockSpec((1, C1, TN), lambda b, t: (b, 0, t)),
            pl.BlockSpec((1, C2, N2), lambda b, t: (b, 0, 0)),
            pl.BlockSpec((CO1, C1), lambda b, t: (0, 0)),
            pl.BlockSpec((CO1, C2), lambda b, t: (0, 0)),
        ],
        out_specs=[
            pl.BlockSpec((1, CO1, TN), lambda b, t: (b, 0, t)),
            pl.BlockSpec((1, CO1, 1), lambda b, t: (b, 0, 0)),
            pl.BlockSpec((1, CO1, 1), lambda b, t: (b, 0, 0)),
        ],
        out_shape=[
            jax.ShapeDtypeStruct((B, CO1, N1), f32),
            jax.ShapeDtypeStruct((B, CO1, 1), f32),
            jax.ShapeDtypeStruct((B, CO1, 1), f32),
        ],
        compiler_params=params,
    )(G2, G1, X1SQ, X2SQ, features1, features2, W1a, W1b)

    count = float(B * N1)
    h2, s2, q2 = pl.pallas_call(
        functools.partial(_stage_b, count=count),
        grid=grid,
        in_specs=[
            pl.BlockSpec((1, CO1, TN), lambda b, t: (b, 0, t)),
            pl.BlockSpec((B, CO1, 1), lambda b, t: (0, 0, 0)),
            pl.BlockSpec((B, CO1, 1), lambda b, t: (0, 0, 0)),
            pl.BlockSpec((CO1, 1), lambda b, t: (0, 0)),
            pl.BlockSpec((CO1, 1), lambda b, t: (0, 0)),
            pl.BlockSpec((CO2, CO1), lambda b, t: (0, 0)),
        ],
        out_specs=[
            pl.BlockSpec((1, CO2, TN), lambda b, t: (b, 0, t)),
            pl.BlockSpec((1, CO2, 1), lambda b, t: (b, 0, 0)),
            pl.BlockSpec((1, CO2, 1), lambda b, t: (b, 0, 0)),
        ],
        out_shape=[
            jax.ShapeDtypeStruct((B, CO2, N1), f32),
            jax.ShapeDtypeStruct((B, CO2, 1), f32),
            jax.ShapeDtypeStruct((B, CO2, 1), f32),
        ],
        compiler_params=params,
    )(h1, s1, q1, g1.reshape(CO1, 1), b1.reshape(CO1, 1), W2)

    out = pl.pallas_call(
        functools.partial(_stage_c, count=count),
        grid=grid,
        in_specs=[
            pl.BlockSpec((1, CO2, TN), lambda b, t: (b, 0, t)),
            pl.BlockSpec((B, CO2, 1), lambda b, t: (0, 0, 0)),
            pl.BlockSpec((B, CO2, 1), lambda b, t: (0, 0, 0)),
            pl.BlockSpec((CO2, 1), lambda b, t: (0, 0)),
            pl.BlockSpec((CO2, 1), lambda b, t: (0, 0)),
        ],
        out_specs=pl.BlockSpec((1, CO2, TN), lambda b, t: (b, 0, t)),
        out_shape=jax.ShapeDtypeStruct((B, CO2, N1), f32),
        compiler_params=params,
    )(h2, s2, q2, g2.reshape(CO2, 1), b2.reshape(CO2, 1))

    return out
